# baseline (speedup 1.0000x reference)
import functools

import jax
import jax.numpy as jnp
from jax import lax
from jax.experimental import pallas as pl
from jax.experimental.pallas import tpu as pltpu
from jax.experimental.pallas import tpu_sc as plsc

NC = 2
NS = 16
NW = NC * NS
CHUNK = 512
NBUF = 4


@functools.partial(jax.jit, static_argnames=("n_chunks", "d"))
def _sc_gather(idx3, table, *, n_chunks, d):
    rows_per_w = n_chunks * CHUNK

    mesh = plsc.VectorSubcoreMesh(
        core_axis_name="c", subcore_axis_name="s",
        num_cores=NC, num_subcores=NS,
    )

    @functools.partial(
        pl.kernel,
        out_type=jax.ShapeDtypeStruct((NW * rows_per_w, d), jnp.float32),
        mesh=mesh,
        scratch_types=[
            pltpu.VMEM((n_chunks, CHUNK), jnp.int32),
            pltpu.VMEM((NBUF, CHUNK, d), jnp.float32),
            pltpu.SemaphoreType.DMA((NBUF,)),
            pltpu.SemaphoreType.DMA((NBUF,)),
        ],
        compiler_params=pltpu.CompilerParams(use_tc_tiling_on_sc=False),
    )
    def body(table_hbm, idx_hbm, out_hbm, idx_v, rows_v, gsem, osem):
        wid = lax.axis_index("s") * NC + lax.axis_index("c")
        base = wid * rows_per_w
        pltpu.sync_copy(idx_hbm.at[wid], idx_v)

        def start_gather(c, b):
            pltpu.async_copy(table_hbm.at[idx_v.at[c]], rows_v.at[b],
                             gsem.at[b])

        def wait_gather(c, b):
            pltpu.make_async_copy(table_hbm.at[idx_v.at[c]], rows_v.at[b],
                                  gsem.at[b]).wait()

        def start_out(c, b):
            pltpu.async_copy(rows_v.at[b],
                             out_hbm.at[pl.ds(base + c * CHUNK, CHUNK)],
                             osem.at[b])

        def wait_out(c, b):
            pltpu.make_async_copy(rows_v.at[b],
                                  out_hbm.at[pl.ds(base + c * CHUNK, CHUNK)],
                                  osem.at[b]).wait()

        for b in range(NBUF):
            start_gather(b, b)

        @pl.loop(0, n_chunks - NBUF, step=NBUF)
        def _(c0):
            for b in range(NBUF):
                c = c0 + b
                wait_gather(c, b)
                start_out(c, b)
                wait_out(c, b)
                start_gather(c + NBUF, b)

        for b in range(NBUF):
            c = n_chunks - NBUF + b
            wait_gather(c, b)
            start_out(c, b)
            wait_out(c, b)

    return body(table, idx3)


def kernel(x, table):
    b, h = x.shape
    v, d = table.shape
    r = b * h
    idx = x.reshape(-1).astype(jnp.int32)

    grain = NW * CHUNK * NBUF
    r_pad = ((r + grain - 1) // grain) * grain
    if r_pad != r:
        idx = jnp.concatenate(
            [idx, jnp.zeros((r_pad - r,), jnp.int32)])
    n_chunks = r_pad // (NW * CHUNK)
    idx3 = idx.reshape(NW, n_chunks, CHUNK)

    out = _sc_gather(idx3, table, n_chunks=n_chunks, d=d)
    return out[:r].reshape(b, h, d)


# speedup vs baseline: 1.6171x; 1.6171x over previous
import functools

import jax
import jax.numpy as jnp
from jax import lax
from jax.experimental import pallas as pl
from jax.experimental.pallas import tpu as pltpu
from jax.experimental.pallas import tpu_sc as plsc

NC = 2
NS = 16
NW = NC * NS
CHUNK = 128
NBUF = 8


@functools.partial(jax.jit, static_argnames=("n_chunks", "d"))
def _sc_gather(idx3, table, *, n_chunks, d):
    rows_per_w = n_chunks * CHUNK

    mesh = plsc.VectorSubcoreMesh(
        core_axis_name="c", subcore_axis_name="s",
        num_cores=NC, num_subcores=NS,
    )

    @functools.partial(
        pl.kernel,
        out_type=jax.ShapeDtypeStruct((NW * rows_per_w, d), jnp.float32),
        mesh=mesh,
        scratch_types=[
            pltpu.VMEM((n_chunks, CHUNK), jnp.int32),
            pltpu.VMEM((NBUF, CHUNK, d), jnp.float32),
            pltpu.SemaphoreType.DMA((NBUF,)),
            pltpu.SemaphoreType.DMA((NBUF,)),
        ],
        compiler_params=pltpu.CompilerParams(use_tc_tiling_on_sc=False),
    )
    def body(table_hbm, idx_hbm, out_hbm, idx_v, rows_v, gsem, osem):
        wid = lax.axis_index("s") * NC + lax.axis_index("c")
        base = wid * rows_per_w
        pltpu.sync_copy(idx_hbm.at[wid], idx_v)

        def start_gather(c, b):
            pltpu.async_copy(table_hbm.at[idx_v.at[c]], rows_v.at[b],
                             gsem.at[b])

        def wait_gather(c, b):
            pltpu.make_async_copy(table_hbm.at[idx_v.at[c]], rows_v.at[b],
                                  gsem.at[b]).wait()

        def start_out(c, b):
            pltpu.async_copy(rows_v.at[b],
                             out_hbm.at[pl.ds(base + c * CHUNK, CHUNK)],
                             osem.at[b])

        def wait_out(c, b):
            pltpu.make_async_copy(rows_v.at[b],
                                  out_hbm.at[pl.ds(base + c * CHUNK, CHUNK)],
                                  osem.at[b]).wait()

        for b in range(NBUF):
            start_gather(b, b)

        @pl.loop(0, n_chunks - NBUF, step=NBUF)
        def _(c0):
            for b in range(NBUF):
                c = c0 + b
                wait_gather(c, b)
                start_out(c, b)
                wait_out(c, b)
                start_gather(c + NBUF, b)

        for b in range(NBUF):
            c = n_chunks - NBUF + b
            wait_gather(c, b)
            start_out(c, b)
            wait_out(c, b)

    return body(table, idx3)


def kernel(x, table):
    b, h = x.shape
    v, d = table.shape
    r = b * h
    idx = x.reshape(-1).astype(jnp.int32)

    grain = NW * CHUNK * NBUF
    r_pad = ((r + grain - 1) // grain) * grain
    if r_pad != r:
        idx = jnp.concatenate(
            [idx, jnp.zeros((r_pad - r,), jnp.int32)])
    n_chunks = r_pad // (NW * CHUNK)
    idx3 = idx.reshape(NW, n_chunks, CHUNK)

    out = _sc_gather(idx3, table, n_chunks=n_chunks, d=d)
    return out[:r].reshape(b, h, d)
